# Initial kernel scaffold; baseline (speedup 1.0000x reference)
import jax
import jax.numpy as jnp
from jax.experimental import pallas as pl


def kernel(seq_len, embeddings):
    raise NotImplementedError("write your pallas kernel here")


# speedup vs baseline: 42.5175x; 42.5175x over previous
import functools

import jax
import jax.numpy as jnp
from jax import lax
from jax.experimental import pallas as pl
from jax.experimental.pallas import tpu as pltpu
from jax.experimental.pallas import tpu_sc as plsc

_MAX_DIST = 128
_H = 16
_S = 2048
_TAB = 2 * _MAX_DIST + 1
_EPAD = 264
_REP = 4112
_PIPE = 16


def _sc_body(emb_t, out, e_row, v8, sem):
    wid = lax.axis_index("s") * 2 + lax.axis_index("c")
    head = wid >> 1
    i0 = (wid & 1) * (_S // 2)

    pltpu.sync_copy(emb_t.at[head], e_row)

    iota = lax.iota(jnp.int32, 16)

    for r in range(8):
        def build(c, _, r=r):
            base = c * 16
            d = base + iota + (r - (_S - 1))
            idx = jnp.clip(d, -_MAX_DIST, _MAX_DIST) + _MAX_DIST
            v8[pl.ds(r * _REP + base, 16)] = plsc.load_gather(e_row, [idx])
            return 0
        lax.fori_loop(0, _REP // 16, build, 0)

    def row(i, _):
        ri = i0 + i
        o = (_S - 1) - ri
        r = o & 7
        start = pl.multiple_of(r * _REP + (o - r), 8)
        pltpu.make_async_copy(v8.at[pl.ds(start, _S)], out.at[head, ri], sem).start()

        @pl.when(i >= _PIPE)
        def _():
            pltpu.make_async_copy(v8.at[pl.ds(0, _S)], out.at[head, i0], sem).wait()

        return 0

    lax.fori_loop(0, _S // 2, row, 0)
    for _ in range(_PIPE):
        pltpu.make_async_copy(v8.at[pl.ds(0, _S)], out.at[head, i0], sem).wait()


@functools.partial(jax.jit)
def _rpb_sc(emb_t):
    mesh = plsc.VectorSubcoreMesh(core_axis_name="c", subcore_axis_name="s")
    f = functools.partial(
        pl.kernel,
        mesh=mesh,
        out_type=jax.ShapeDtypeStruct((_H, _S, _S), jnp.float32),
        compiler_params=pltpu.CompilerParams(
            needs_layout_passes=False, use_tc_tiling_on_sc=False
        ),
        scratch_types=[
            pltpu.VMEM((_EPAD,), jnp.float32),
            pltpu.VMEM((8 * _REP,), jnp.float32),
            pltpu.SemaphoreType.DMA,
        ],
    )(_sc_body)
    return f(emb_t)


def kernel(seq_len, embeddings):
    del seq_len
    emb_t = jnp.zeros((_H, _EPAD), jnp.float32).at[:, :_TAB].set(embeddings.T)
    return _rpb_sc(emb_t)
